# Initial kernel scaffold; baseline (speedup 1.0000x reference)
import functools

import jax
import jax.numpy as jnp
from jax import lax
from jax.experimental import pallas as pl
from jax.experimental.pallas import tpu as pltpu
from jax.experimental.pallas import tpu_sc as plsc

N_NODES = 10000
F = 128
NC = 2
NS = 16
NW = NC * NS
CHUNK = 80
DUMMY_ROWS = 16
DEG_W = 8


def _sc_mesh():
    return plsc.VectorSubcoreMesh(
        core_axis_name="c", subcore_axis_name="s", num_cores=NC, num_subcores=NS
    )


def _make_deg_kernel(n_nodes, g_chunks):
    mesh = _sc_mesh()

    @functools.partial(
        pl.kernel,
        out_type=jax.ShapeDtypeStruct((NC, n_nodes, DEG_W), jnp.float32),
        mesh=mesh,
        scratch_types=[
            pltpu.VMEM((g_chunks, CHUNK), jnp.int32),
            pltpu.VMEM((CHUNK, DEG_W), jnp.float32),
            pltpu.VMEM_SHARED((n_nodes + DUMMY_ROWS, DEG_W), jnp.float32),
        ],
    )
    def deg_kernel(dst_hbm, ones_hbm, zeros_hbm, out_hbm, dst_v, ones_v, table):
        cid = lax.axis_index("c")
        sid = lax.axis_index("s")
        wid = cid * NS + sid

        rows_per_tile = n_nodes // NS
        sl = pl.ds(sid * rows_per_tile, rows_per_tile)
        pltpu.sync_copy(zeros_hbm.at[sl], table.at[sl])
        pltpu.sync_copy(ones_hbm, ones_v)
        pltpu.sync_copy(dst_hbm.at[wid], dst_v)
        plsc.subcore_barrier()

        def body(g, carry):
            pltpu.sync_copy(ones_v, table.at[dst_v.at[g]], add=True)
            return carry

        lax.fori_loop(0, g_chunks, body, 0)
        plsc.subcore_barrier()

        @pl.when(sid == 0)
        def _():
            pltpu.sync_copy(table.at[pl.ds(0, n_nodes)], out_hbm.at[cid])

    return deg_kernel


def _make_agg_kernel(n_nodes, g_chunks):
    mesh = _sc_mesh()

    @functools.partial(
        pl.kernel,
        out_type=jax.ShapeDtypeStruct((NC, n_nodes, F), jnp.float32),
        mesh=mesh,
        scratch_types=[
            pltpu.VMEM((g_chunks, CHUNK), jnp.int32),
            pltpu.VMEM((g_chunks, CHUNK), jnp.int32),
            pltpu.VMEM((CHUNK, F), jnp.float32),
            pltpu.VMEM_SHARED((n_nodes + DUMMY_ROWS, F), jnp.float32),
        ],
    )
    def agg_kernel(hp_hbm, src_hbm, dst_hbm, zeros_hbm, out_hbm,
                   src_v, dst_v, rows_v, acc):
        cid = lax.axis_index("c")
        sid = lax.axis_index("s")
        wid = cid * NS + sid

        rows_per_tile = n_nodes // NS
        sl = pl.ds(sid * rows_per_tile, rows_per_tile)

        @pl.when(cid == 0)
        def _():
            pltpu.sync_copy(hp_hbm.at[sl], acc.at[sl])

        @pl.when(cid != 0)
        def _():
            pltpu.sync_copy(zeros_hbm.at[sl], acc.at[sl])

        pltpu.sync_copy(src_hbm.at[wid], src_v)
        pltpu.sync_copy(dst_hbm.at[wid], dst_v)
        plsc.subcore_barrier()

        def body(g, carry):
            pltpu.sync_copy(hp_hbm.at[src_v.at[g]], rows_v)
            pltpu.sync_copy(rows_v, acc.at[dst_v.at[g]], add=True)
            return carry

        lax.fori_loop(0, g_chunks, body, 0)
        plsc.subcore_barrier()

        pltpu.sync_copy(acc.at[sl], out_hbm.at[cid, sl])

    return agg_kernel


BLK = 1000


def _dinv(d0_ref, d1_ref):
    deg = d0_ref[:, 0:1] + d1_ref[:, 0:1] + 1.0
    return lax.rsqrt(deg)


def _stage_a_body(d0_ref, d1_ref, x_ref, w_ref, hp_ref):
    dinv = _dinv(d0_ref, d1_ref)
    hp_ref[...] = jnp.dot(
        x_ref[...] * dinv, w_ref[...], preferred_element_type=jnp.float32
    )


def _stage_b_body(d0_ref, d1_ref, a0_ref, a1_ref, b_ref, w_ref, hp_ref):
    dinv = _dinv(d0_ref, d1_ref)
    s = a0_ref[...] + a1_ref[...]
    t = jnp.maximum(dinv * s + b_ref[...], 0.0)
    hp_ref[...] = jnp.dot(
        t * dinv, w_ref[...], preferred_element_type=jnp.float32
    )


def _stage_c_body(d0_ref, d1_ref, a0_ref, a1_ref, b_ref, w_ref, bc_ref, o_ref):
    dinv = _dinv(d0_ref, d1_ref)
    s = a0_ref[...] + a1_ref[...]
    t = jnp.maximum(dinv * s + b_ref[...], 0.0)
    o_ref[...] = jnp.dot(
        t, w_ref[...], preferred_element_type=jnp.float32
    ) + bc_ref[...]


def _row_spec(width):
    return pl.BlockSpec((BLK, width), lambda i: (i, 0))


def _full_spec(shape):
    return pl.BlockSpec(shape, lambda i: tuple(0 for _ in shape))


def _stage_a(d0, d1, x, w):
    n = x.shape[0]
    return pl.pallas_call(
        _stage_a_body,
        grid=(n // BLK,),
        in_specs=[_row_spec(DEG_W), _row_spec(DEG_W), _row_spec(F),
                  _full_spec((F, F))],
        out_specs=_row_spec(F),
        out_shape=jax.ShapeDtypeStruct((n, F), jnp.float32),
    )(d0, d1, x, w)


def _stage_b(d0, d1, a0, a1, b, w):
    n = a0.shape[0]
    return pl.pallas_call(
        _stage_b_body,
        grid=(n // BLK,),
        in_specs=[_row_spec(DEG_W), _row_spec(DEG_W), _row_spec(F),
                  _row_spec(F), _full_spec((1, F)), _full_spec((F, F))],
        out_specs=_row_spec(F),
        out_shape=jax.ShapeDtypeStruct((n, F), jnp.float32),
    )(d0, d1, a0, a1, b, w)


def _stage_c(d0, d1, a0, a1, b, w, bc):
    n = a0.shape[0]
    k = w.shape[1]
    return pl.pallas_call(
        _stage_c_body,
        grid=(n // BLK,),
        in_specs=[_row_spec(DEG_W), _row_spec(DEG_W), _row_spec(F),
                  _row_spec(F), _full_spec((1, F)), _full_spec((F, k)),
                  _full_spec((1, k))],
        out_specs=pl.BlockSpec((BLK, k), lambda i: (i, 0)),
        out_shape=jax.ShapeDtypeStruct((n, k), jnp.float32),
    )(d0, d1, a0, a1, b, w, bc)


def kernel(x, edge_index, W1, b1, W2, b2, Wc, bc):
    n = x.shape[0]
    src = edge_index[0].astype(jnp.int32)
    dst = edge_index[1].astype(jnp.int32)
    e = src.shape[0]

    per = NW * CHUNK
    g_chunks = -(-e // per)
    e_pad = g_chunks * per
    if e_pad != e:
        src = jnp.concatenate(
            [src, jnp.zeros((e_pad - e,), jnp.int32)])
        dst = jnp.concatenate(
            [dst, jnp.full((e_pad - e,), n, jnp.int32)])
    src_r = src.reshape(NW, g_chunks, CHUNK)
    dst_r = dst.reshape(NW, g_chunks, CHUNK)

    zeros2 = jnp.zeros((n, F), jnp.float32)
    zeros_deg = jnp.zeros((n, DEG_W), jnp.float32)
    ones_deg = jnp.ones((CHUNK, DEG_W), jnp.float32)

    deg_kernel = _make_deg_kernel(n, g_chunks)
    agg_kernel = _make_agg_kernel(n, g_chunks)

    degp = deg_kernel(dst_r, ones_deg, zeros_deg)
    d0, d1 = degp[0], degp[1]

    hp1 = _stage_a(d0, d1, x, W1)
    acc1 = agg_kernel(hp1, src_r, dst_r, zeros2)
    hp2 = _stage_b(d0, d1, acc1[0], acc1[1], b1.reshape(1, F), W2)
    acc2 = agg_kernel(hp2, src_r, dst_r, zeros2)
    out = _stage_c(d0, d1, acc2[0], acc2[1], b2.reshape(1, F), Wc,
                   bc.reshape(1, -1))
    return out


# speedup vs baseline: 17.0665x; 17.0665x over previous
import functools

import jax
import jax.numpy as jnp
from jax import lax
from jax.experimental import pallas as pl
from jax.experimental.pallas import tpu as pltpu
from jax.experimental.pallas import tpu_sc as plsc

N_NODES = 10000
F = 128
NC = 2
NS = 16
NW = NC * NS
CHUNK = 80
DUMMY_ROWS = 16
DEG_W = 8


def _sc_mesh():
    return plsc.VectorSubcoreMesh(
        core_axis_name="c", subcore_axis_name="s", num_cores=NC, num_subcores=NS
    )


def _make_deg_kernel(n_nodes, g_chunks):
    mesh = _sc_mesh()

    @functools.partial(
        pl.kernel,
        out_type=jax.ShapeDtypeStruct((NC, n_nodes, F), jnp.float32),
        mesh=mesh,
        scratch_types=[
            pltpu.VMEM((g_chunks, CHUNK), jnp.int32),
            pltpu.VMEM((CHUNK, F), jnp.float32),
            pltpu.VMEM_SHARED((n_nodes + DUMMY_ROWS, F), jnp.float32),
        ],
    )
    def deg_kernel(dst_hbm, ones_hbm, zeros_hbm, out_hbm, dst_v, ones_v, table):
        cid = lax.axis_index("c")
        sid = lax.axis_index("s")
        wid = cid * NS + sid

        rpt = (n_nodes // NS) // 8 * 8
        rem = n_nodes - rpt * NS
        sl = pl.ds(sid * rpt, rpt)
        pltpu.sync_copy(zeros_hbm.at[sl], table.at[sl])
        if rem:
            @pl.when(sid == NS - 1)
            def _():
                rsl = pl.ds(NS * rpt, rem)
                pltpu.sync_copy(zeros_hbm.at[rsl], table.at[rsl])
        pltpu.sync_copy(ones_hbm, ones_v)
        pltpu.sync_copy(dst_hbm.at[wid], dst_v)
        plsc.subcore_barrier()

        def body(g, carry):
            pltpu.sync_copy(ones_v, table.at[dst_v.at[g]], add=True)
            return carry

        lax.fori_loop(0, g_chunks, body, 0)
        plsc.subcore_barrier()

        @pl.when(sid == 0)
        def _():
            pltpu.sync_copy(table.at[pl.ds(0, n_nodes)], out_hbm.at[cid])

    return deg_kernel


def _make_agg_kernel(n_nodes, g_chunks):
    mesh = _sc_mesh()

    @functools.partial(
        pl.kernel,
        out_type=jax.ShapeDtypeStruct((NC, n_nodes, F), jnp.float32),
        mesh=mesh,
        scratch_types=[
            pltpu.VMEM((g_chunks, CHUNK), jnp.int32),
            pltpu.VMEM((g_chunks, CHUNK), jnp.int32),
            pltpu.VMEM((CHUNK, F), jnp.float32),
            pltpu.VMEM_SHARED((n_nodes + DUMMY_ROWS, F), jnp.float32),
        ],
    )
    def agg_kernel(hp_hbm, src_hbm, dst_hbm, zeros_hbm, out_hbm,
                   src_v, dst_v, rows_v, acc):
        cid = lax.axis_index("c")
        sid = lax.axis_index("s")
        wid = cid * NS + sid

        rpt = (n_nodes // NS) // 8 * 8
        rem = n_nodes - rpt * NS
        sl = pl.ds(sid * rpt, rpt)
        rsl = pl.ds(NS * rpt, rem)

        @pl.when(cid == 0)
        def _():
            pltpu.sync_copy(hp_hbm.at[sl], acc.at[sl])

        @pl.when(cid != 0)
        def _():
            pltpu.sync_copy(zeros_hbm.at[sl], acc.at[sl])

        if rem:
            @pl.when(sid == NS - 1)
            def _():
                @pl.when(cid == 0)
                def _():
                    pltpu.sync_copy(hp_hbm.at[rsl], acc.at[rsl])

                @pl.when(cid != 0)
                def _():
                    pltpu.sync_copy(zeros_hbm.at[rsl], acc.at[rsl])

        pltpu.sync_copy(src_hbm.at[wid], src_v)
        pltpu.sync_copy(dst_hbm.at[wid], dst_v)
        plsc.subcore_barrier()

        def body(g, carry):
            pltpu.sync_copy(hp_hbm.at[src_v.at[g]], rows_v)
            pltpu.sync_copy(rows_v, acc.at[dst_v.at[g]], add=True)
            return carry

        lax.fori_loop(0, g_chunks, body, 0)
        plsc.subcore_barrier()

        pltpu.sync_copy(acc.at[sl], out_hbm.at[cid, sl])
        if rem:
            @pl.when(sid == NS - 1)
            def _():
                pltpu.sync_copy(acc.at[rsl], out_hbm.at[cid, rsl])

    return agg_kernel


BLK = 1000


def _dinv(d0_ref, d1_ref):
    deg = d0_ref[:, 0:1] + d1_ref[:, 0:1] + 1.0
    return lax.rsqrt(deg)


def _stage_a_body(d0_ref, d1_ref, x_ref, w_ref, hp_ref):
    dinv = _dinv(d0_ref, d1_ref)
    hp_ref[...] = jnp.dot(
        x_ref[...] * dinv, w_ref[...], preferred_element_type=jnp.float32
    )


def _stage_b_body(d0_ref, d1_ref, a0_ref, a1_ref, b_ref, w_ref, hp_ref):
    dinv = _dinv(d0_ref, d1_ref)
    s = a0_ref[...] + a1_ref[...]
    t = jnp.maximum(dinv * s + b_ref[...], 0.0)
    hp_ref[...] = jnp.dot(
        t * dinv, w_ref[...], preferred_element_type=jnp.float32
    )


def _stage_c_body(d0_ref, d1_ref, a0_ref, a1_ref, b_ref, w_ref, bc_ref, o_ref):
    dinv = _dinv(d0_ref, d1_ref)
    s = a0_ref[...] + a1_ref[...]
    t = jnp.maximum(dinv * s + b_ref[...], 0.0)
    o_ref[...] = jnp.dot(
        t, w_ref[...], preferred_element_type=jnp.float32
    ) + bc_ref[...]


def _row_spec(width):
    return pl.BlockSpec((BLK, width), lambda i: (i, 0))


def _full_spec(shape):
    return pl.BlockSpec(shape, lambda i: tuple(0 for _ in shape))


def _stage_a(d0, d1, x, w):
    n = x.shape[0]
    return pl.pallas_call(
        _stage_a_body,
        grid=(n // BLK,),
        in_specs=[_row_spec(DEG_W), _row_spec(DEG_W), _row_spec(F),
                  _full_spec((F, F))],
        out_specs=_row_spec(F),
        out_shape=jax.ShapeDtypeStruct((n, F), jnp.float32),
    )(d0, d1, x, w)


def _stage_b(d0, d1, a0, a1, b, w):
    n = a0.shape[0]
    return pl.pallas_call(
        _stage_b_body,
        grid=(n // BLK,),
        in_specs=[_row_spec(DEG_W), _row_spec(DEG_W), _row_spec(F),
                  _row_spec(F), _full_spec((1, F)), _full_spec((F, F))],
        out_specs=_row_spec(F),
        out_shape=jax.ShapeDtypeStruct((n, F), jnp.float32),
    )(d0, d1, a0, a1, b, w)


def _stage_c(d0, d1, a0, a1, b, w, bc):
    n = a0.shape[0]
    k = w.shape[1]
    return pl.pallas_call(
        _stage_c_body,
        grid=(n // BLK,),
        in_specs=[_row_spec(DEG_W), _row_spec(DEG_W), _row_spec(F),
                  _row_spec(F), _full_spec((1, F)), _full_spec((F, k)),
                  _full_spec((1, k))],
        out_specs=pl.BlockSpec((BLK, k), lambda i: (i, 0)),
        out_shape=jax.ShapeDtypeStruct((n, k), jnp.float32),
    )(d0, d1, a0, a1, b, w, bc)


def kernel(x, edge_index, W1, b1, W2, b2, Wc, bc):
    n = x.shape[0]
    src = edge_index[0].astype(jnp.int32)
    dst = edge_index[1].astype(jnp.int32)
    e = src.shape[0]

    per = NW * CHUNK
    g_chunks = -(-e // per)
    e_pad = g_chunks * per
    if e_pad != e:
        src = jnp.concatenate(
            [src, jnp.zeros((e_pad - e,), jnp.int32)])
        dst = jnp.concatenate(
            [dst, jnp.full((e_pad - e,), n, jnp.int32)])
    src_r = src.reshape(NW, g_chunks, CHUNK)
    dst_r = dst.reshape(NW, g_chunks, CHUNK)

    zeros2 = jnp.zeros((n, F), jnp.float32)
    ones_deg = jnp.ones((CHUNK, F), jnp.float32)

    deg_kernel = _make_deg_kernel(n, g_chunks)
    agg_kernel = _make_agg_kernel(n, g_chunks)

    degp = deg_kernel(dst_r, ones_deg, zeros2)
    d0, d1 = degp[0, :, :DEG_W], degp[1, :, :DEG_W]

    hp1 = _stage_a(d0, d1, x, W1)
    acc1 = agg_kernel(hp1, src_r, dst_r, zeros2)
    hp2 = _stage_b(d0, d1, acc1[0], acc1[1], b1.reshape(1, F), W2)
    acc2 = agg_kernel(hp2, src_r, dst_r, zeros2)
    out = _stage_c(d0, d1, acc2[0], acc2[1], b2.reshape(1, F), Wc,
                   bc.reshape(1, -1))
    return out
